# baseline (speedup 1.0000x reference)
import functools

import jax
import jax.numpy as jnp
from jax.experimental import pallas as pl
from jax.experimental.pallas import tpu as pltpu

_NFEAT = 128
_NHID = 64
_NCLASS = 32
_B = 2

_ROW_BLK = 256
_L2_BLK = 512


def _feat_body(n_sets, w1_ref, wf_ref, bf_ref, *refs):
    x_refs = refs[:n_sets]
    h1_ref, init_ref = refs[n_sets], refs[n_sets + 1]
    w1 = w1_ref[...]
    wf = wf_ref[...]
    bf = bf_ref[...]
    h1_parts = []
    init_parts = []
    for x_ref in x_refs:
        for b in range(_B):
            xb = x_ref[b]
            h1_parts.append(
                jnp.dot(xb, w1, preferred_element_type=jnp.float32))
            init_parts.append(
                jnp.dot(xb, wf, preferred_element_type=jnp.float32) + bf)
    h1_ref[...] = jnp.concatenate(h1_parts, axis=1)
    init_ref[...] = jnp.concatenate(init_parts, axis=1)


def _feat(xs, conv1_w, fc1_w, fc1_b):
    n_sets = len(xs)
    n = xs[0].shape[1]
    grid = pl.cdiv(n, _ROW_BLK)
    return pl.pallas_call(
        functools.partial(_feat_body, n_sets),
        grid=(grid,),
        in_specs=[
            pl.BlockSpec((_NFEAT, _NHID), lambda i: (0, 0)),
            pl.BlockSpec((_NFEAT, _NCLASS), lambda i: (0, 0)),
            pl.BlockSpec((1, _NCLASS), lambda i: (0, 0)),
        ] + [
            pl.BlockSpec((_B, _ROW_BLK, _NFEAT), lambda i: (0, i, 0))
            for _ in range(n_sets)
        ],
        out_specs=[
            pl.BlockSpec((_ROW_BLK, n_sets * _B * _NHID), lambda i: (i, 0)),
            pl.BlockSpec((_ROW_BLK, n_sets * _B * _NCLASS), lambda i: (i, 0)),
        ],
        out_shape=[
            jax.ShapeDtypeStruct((n, n_sets * _B * _NHID), jnp.float32),
            jax.ShapeDtypeStruct((n, n_sets * _B * _NCLASS), jnp.float32),
        ],
    )(conv1_w, fc1_w, fc1_b.reshape(1, _NCLASS), *xs)


def _act(t, nmf_span):
    if nmf_span is None:
        return jnp.maximum(t, 0.0)
    lo, hi = nmf_span
    col = jax.lax.broadcasted_iota(jnp.int32, t.shape, 1)
    keep_linear = (col >= lo) & (col < hi)
    return jnp.where(keep_linear, t, jnp.maximum(t, 0.0))


def _layer1_body(nmf_span, adj_ref, h1_ref, b1_ref, w2_ref, h2_ref,
                 adj16_ref):
    a = adj_ref[...]
    adj16_ref[...] = a.astype(jnp.bfloat16)
    s = _act(jnp.dot(a, h1_ref[...],
                     preferred_element_type=jnp.float32) + b1_ref[...],
             nmf_span)
    h2_ref[...] = jnp.dot(s, w2_ref[...], preferred_element_type=jnp.float32)


def _layer2_body(n_rows, adj16_ref, h2_ref, init_ref, b2_ref, pool_ref):
    i = pl.program_id(0)
    t = (
        jnp.dot(adj16_ref[...], h2_ref[...],
                preferred_element_type=jnp.float32)
        + b2_ref[...] + init_ref[...]
    )
    rows = jax.lax.broadcasted_iota(jnp.int32, t.shape, 0) + i * _L2_BLK
    t = jnp.where(rows < n_rows, t, -jnp.inf)
    m = jnp.max(t, axis=0, keepdims=True)
    m8 = jnp.broadcast_to(m, (8, t.shape[1]))

    @pl.when(i == 0)
    def _():
        pool_ref[...] = m8

    @pl.when(i > 0)
    def _():
        pool_ref[...] = jnp.maximum(pool_ref[...], m8)


def _gcn_pool(adj, h1, init, b1, b2, w2d, nmf_span):
    n = adj.shape[0]
    f1 = h1.shape[1]
    f2 = w2d.shape[1]
    params = pltpu.CompilerParams(vmem_limit_bytes=64 * 1024 * 1024)
    h2, adj16 = pl.pallas_call(
        functools.partial(_layer1_body, nmf_span),
        grid=(pl.cdiv(n, _ROW_BLK),),
        in_specs=[
            pl.BlockSpec((_ROW_BLK, n), lambda i: (i, 0)),
            pl.BlockSpec((n, f1), lambda i: (0, 0)),
            pl.BlockSpec((1, f1), lambda i: (0, 0)),
            pl.BlockSpec((f1, f2), lambda i: (0, 0)),
        ],
        out_specs=[
            pl.BlockSpec((_ROW_BLK, f2), lambda i: (i, 0)),
            pl.BlockSpec((_ROW_BLK, n), lambda i: (i, 0)),
        ],
        out_shape=[
            jax.ShapeDtypeStruct((n, f2), jnp.float32),
            jax.ShapeDtypeStruct((n, n), jnp.bfloat16),
        ],
        compiler_params=params,
    )(adj, h1, b1, w2d)
    pooled = pl.pallas_call(
        functools.partial(_layer2_body, n),
        grid=(pl.cdiv(n, _L2_BLK),),
        in_specs=[
            pl.BlockSpec((_L2_BLK, n), lambda i: (i, 0)),
            pl.BlockSpec((n, f2), lambda i: (0, 0)),
            pl.BlockSpec((_L2_BLK, f2), lambda i: (i, 0)),
            pl.BlockSpec((1, f2), lambda i: (0, 0)),
        ],
        out_specs=pl.BlockSpec((8, f2), lambda i: (0, 0)),
        out_shape=jax.ShapeDtypeStruct((8, f2), jnp.float32),
        compiler_params=params,
    )(adj16, h2, init, b2)
    return pooled[0]


def _head_body(x_ref, w2_ref, b2_ref, w3_ref, b3_ref, w4_ref, b4_ref,
               w5_ref, b5_ref, out_ref):
    x = x_ref[...]
    x = jnp.maximum(jnp.dot(x, w2_ref[...], preferred_element_type=jnp.float32)
                    + b2_ref[...], 0.0)
    x = jnp.maximum(jnp.dot(x, w3_ref[...], preferred_element_type=jnp.float32)
                    + b3_ref[...], 0.0)
    x = jnp.maximum(jnp.dot(x, w4_ref[...], preferred_element_type=jnp.float32)
                    + b4_ref[...], 0.0)
    d = jnp.dot(x, w5_ref[...], preferred_element_type=jnp.float32) + b5_ref[...]
    out_ref[...] = d + jnp.zeros((8, 8), jnp.float32)


def _head(x8, fc2_w, fc2_b, fc3_w, fc3_b, fc4_w, fc4_b, fc5_w, fc5_b):
    full = lambda shape: pl.BlockSpec(shape, lambda: (0,) * len(shape))
    return pl.pallas_call(
        _head_body,
        in_specs=[
            full((8, 2 * _NCLASS)),
            full(fc2_w.shape), full((1, _NCLASS)),
            full(fc3_w.shape), full((1, 64)),
            full(fc4_w.shape), full((1, 32)),
            full(fc5_w.shape), full((1, 1)),
        ],
        out_specs=full((8, 8)),
        out_shape=jax.ShapeDtypeStruct((8, 8), jnp.float32),
    )(x8, fc2_w, fc2_b.reshape(1, -1), fc3_w, fc3_b.reshape(1, -1),
      fc4_w, fc4_b.reshape(1, -1), fc5_w, fc5_b.reshape(1, -1))


def kernel(solute_ACE, solvent_ACE, solute_adj, solvent_adj_ACE, solute_NMF,
           solvent_NMF, solvent_adj_NMF, solute_wat, solvent_wat,
           solvent_adj_wat, fc1_w, fc1_b, conv1_w, conv1_b, conv2_w, conv2_b,
           fc2_w, fc2_b, fc3_w, fc3_b, fc4_w, fc4_b, fc5_w, fc5_b):
    su_h1, su_init = _feat([solute_ACE, solute_NMF, solute_wat],
                           conv1_w, fc1_w, fc1_b)
    sv_h1, sv_init = {}, {}
    sv_h1['ACE'], sv_init['ACE'] = _feat([solvent_ACE], conv1_w, fc1_w, fc1_b)
    sv_h1['NMF'], sv_init['NMF'] = _feat([solvent_NMF], conv1_w, fc1_w, fc1_b)
    sv_h1['wat'], sv_init['wat'] = _feat([solvent_wat], conv1_w, fc1_w, fc1_b)

    b1_sv = jnp.tile(conv1_b, _B).reshape(1, -1)
    b2_sv = jnp.tile(conv2_b, _B).reshape(1, -1)
    b1_su = jnp.tile(conv1_b, 3 * _B).reshape(1, -1)
    b2_su = jnp.tile(conv2_b, 3 * _B).reshape(1, -1)
    w2_sv = jnp.kron(jnp.eye(_B, dtype=jnp.float32), conv2_w)
    w2_su = jnp.kron(jnp.eye(3 * _B, dtype=jnp.float32), conv2_w)

    p_su = _gcn_pool(solute_adj, su_h1, su_init, b1_su, b2_su, w2_su,
                     nmf_span=(_B * _NHID, 2 * _B * _NHID))
    p_ace = _gcn_pool(solvent_adj_ACE, sv_h1['ACE'], sv_init['ACE'],
                      b1_sv, b2_sv, w2_sv, nmf_span=None)
    p_nmf = _gcn_pool(solvent_adj_NMF, sv_h1['NMF'], sv_init['NMF'],
                      b1_sv, b2_sv, w2_sv, nmf_span=None)
    p_wat = _gcn_pool(solvent_adj_wat, sv_h1['wat'], sv_init['wat'],
                      b1_sv, b2_sv, w2_sv, nmf_span=None)

    c = _NCLASS
    rows = [
        jnp.concatenate([p_su[0 * c:1 * c], p_ace[0:c]]),
        jnp.concatenate([p_su[1 * c:2 * c], p_ace[c:2 * c]]),
        jnp.concatenate([p_su[2 * c:3 * c], p_nmf[0:c]]),
        jnp.concatenate([p_su[3 * c:4 * c], p_nmf[c:2 * c]]),
        jnp.concatenate([p_su[4 * c:5 * c], p_wat[0:c]]),
        jnp.concatenate([p_su[5 * c:6 * c], p_wat[c:2 * c]]),
    ]
    x8 = jnp.pad(jnp.stack(rows), ((0, 2), (0, 0)))
    out = _head(x8, fc2_w, fc2_b, fc3_w, fc3_b, fc4_w, fc4_b, fc5_w, fc5_b)
    return out[:6, :1]


# speedup vs baseline: 1.0559x; 1.0559x over previous
import functools

import jax
import jax.numpy as jnp
from jax.experimental import pallas as pl
from jax.experimental.pallas import tpu as pltpu

_NFEAT = 128
_NHID = 64
_NCLASS = 32
_B = 2

_ROW_BLK = 384
_L2_BLK = 384


def _feat_body(n_sets, w1_ref, wf_ref, bf_ref, *refs):
    x_refs = refs[:n_sets]
    h1_ref, init_ref = refs[n_sets], refs[n_sets + 1]
    w1 = w1_ref[...]
    wf = wf_ref[...]
    bf = bf_ref[...]
    h1_parts = []
    init_parts = []
    for x_ref in x_refs:
        for b in range(_B):
            xb = x_ref[b]
            h1_parts.append(
                jnp.dot(xb, w1, preferred_element_type=jnp.float32))
            init_parts.append(
                jnp.dot(xb, wf, preferred_element_type=jnp.float32) + bf)
    h1_ref[...] = jnp.concatenate(h1_parts, axis=1)
    init_ref[...] = jnp.concatenate(init_parts, axis=1)


def _feat(xs, conv1_w, fc1_w, fc1_b):
    n_sets = len(xs)
    n = xs[0].shape[1]
    grid = pl.cdiv(n, _ROW_BLK)
    return pl.pallas_call(
        functools.partial(_feat_body, n_sets),
        grid=(grid,),
        in_specs=[
            pl.BlockSpec((_NFEAT, _NHID), lambda i: (0, 0)),
            pl.BlockSpec((_NFEAT, _NCLASS), lambda i: (0, 0)),
            pl.BlockSpec((1, _NCLASS), lambda i: (0, 0)),
        ] + [
            pl.BlockSpec((_B, _ROW_BLK, _NFEAT), lambda i: (0, i, 0))
            for _ in range(n_sets)
        ],
        out_specs=[
            pl.BlockSpec((_ROW_BLK, n_sets * _B * _NHID), lambda i: (i, 0)),
            pl.BlockSpec((_ROW_BLK, n_sets * _B * _NCLASS), lambda i: (i, 0)),
        ],
        out_shape=[
            jax.ShapeDtypeStruct((n, n_sets * _B * _NHID), jnp.float32),
            jax.ShapeDtypeStruct((n, n_sets * _B * _NCLASS), jnp.float32),
        ],
    )(conv1_w, fc1_w, fc1_b.reshape(1, _NCLASS), *xs)


def _act(t, nmf_span):
    if nmf_span is None:
        return jnp.maximum(t, 0.0)
    lo, hi = nmf_span
    col = jax.lax.broadcasted_iota(jnp.int32, t.shape, 1)
    keep_linear = (col >= lo) & (col < hi)
    return jnp.where(keep_linear, t, jnp.maximum(t, 0.0))


def _layer1_body(nmf_span, adj_ref, h1_ref, b1_ref, w2_ref, h2_ref):
    s = _act(jnp.dot(adj_ref[...], h1_ref[...],
                     preferred_element_type=jnp.float32) + b1_ref[...],
             nmf_span)
    h2_ref[...] = jnp.dot(s, w2_ref[...], preferred_element_type=jnp.float32)


def _layer2_body(n_rows, adj_ref, h2_ref, init_ref, b2_ref, pool_ref):
    i = pl.program_id(0)
    t = (
        jnp.dot(adj_ref[...], h2_ref[...],
                preferred_element_type=jnp.float32)
        + b2_ref[...] + init_ref[...]
    )
    rows = jax.lax.broadcasted_iota(jnp.int32, t.shape, 0) + i * _L2_BLK
    t = jnp.where(rows < n_rows, t, -jnp.inf)
    m = jnp.max(t, axis=0, keepdims=True)
    m8 = jnp.broadcast_to(m, (8, t.shape[1]))

    @pl.when(i == 0)
    def _():
        pool_ref[...] = m8

    @pl.when(i > 0)
    def _():
        pool_ref[...] = jnp.maximum(pool_ref[...], m8)


def _gcn_pool(adj, h1, init, b1, b2, w2d, nmf_span):
    n = adj.shape[0]
    f1 = h1.shape[1]
    f2 = w2d.shape[1]
    params = pltpu.CompilerParams(vmem_limit_bytes=64 * 1024 * 1024)
    h2 = pl.pallas_call(
        functools.partial(_layer1_body, nmf_span),
        grid=(pl.cdiv(n, _ROW_BLK),),
        in_specs=[
            pl.BlockSpec((_ROW_BLK, n), lambda i: (i, 0)),
            pl.BlockSpec((n, f1), lambda i: (0, 0)),
            pl.BlockSpec((1, f1), lambda i: (0, 0)),
            pl.BlockSpec((f1, f2), lambda i: (0, 0)),
        ],
        out_specs=pl.BlockSpec((_ROW_BLK, f2), lambda i: (i, 0)),
        out_shape=jax.ShapeDtypeStruct((n, f2), jnp.float32),
        compiler_params=params,
    )(adj, h1, b1, w2d)
    pooled = pl.pallas_call(
        functools.partial(_layer2_body, n),
        grid=(pl.cdiv(n, _L2_BLK),),
        in_specs=[
            pl.BlockSpec((_L2_BLK, n), lambda i: (i, 0)),
            pl.BlockSpec((n, f2), lambda i: (0, 0)),
            pl.BlockSpec((_L2_BLK, f2), lambda i: (i, 0)),
            pl.BlockSpec((1, f2), lambda i: (0, 0)),
        ],
        out_specs=pl.BlockSpec((8, f2), lambda i: (0, 0)),
        out_shape=jax.ShapeDtypeStruct((8, f2), jnp.float32),
        compiler_params=params,
    )(adj, h2, init, b2)
    return pooled[0]


def _head_body(x_ref, w2_ref, b2_ref, w3_ref, b3_ref, w4_ref, b4_ref,
               w5_ref, b5_ref, out_ref):
    x = x_ref[...]
    x = jnp.maximum(jnp.dot(x, w2_ref[...], preferred_element_type=jnp.float32)
                    + b2_ref[...], 0.0)
    x = jnp.maximum(jnp.dot(x, w3_ref[...], preferred_element_type=jnp.float32)
                    + b3_ref[...], 0.0)
    x = jnp.maximum(jnp.dot(x, w4_ref[...], preferred_element_type=jnp.float32)
                    + b4_ref[...], 0.0)
    d = jnp.dot(x, w5_ref[...], preferred_element_type=jnp.float32) + b5_ref[...]
    out_ref[...] = d + jnp.zeros((8, 8), jnp.float32)


def _head(x8, fc2_w, fc2_b, fc3_w, fc3_b, fc4_w, fc4_b, fc5_w, fc5_b):
    full = lambda shape: pl.BlockSpec(shape, lambda: (0,) * len(shape))
    return pl.pallas_call(
        _head_body,
        in_specs=[
            full((8, 2 * _NCLASS)),
            full(fc2_w.shape), full((1, _NCLASS)),
            full(fc3_w.shape), full((1, 64)),
            full(fc4_w.shape), full((1, 32)),
            full(fc5_w.shape), full((1, 1)),
        ],
        out_specs=full((8, 8)),
        out_shape=jax.ShapeDtypeStruct((8, 8), jnp.float32),
    )(x8, fc2_w, fc2_b.reshape(1, -1), fc3_w, fc3_b.reshape(1, -1),
      fc4_w, fc4_b.reshape(1, -1), fc5_w, fc5_b.reshape(1, -1))


def kernel(solute_ACE, solvent_ACE, solute_adj, solvent_adj_ACE, solute_NMF,
           solvent_NMF, solvent_adj_NMF, solute_wat, solvent_wat,
           solvent_adj_wat, fc1_w, fc1_b, conv1_w, conv1_b, conv2_w, conv2_b,
           fc2_w, fc2_b, fc3_w, fc3_b, fc4_w, fc4_b, fc5_w, fc5_b):
    su_h1, su_init = _feat([solute_ACE, solute_NMF, solute_wat],
                           conv1_w, fc1_w, fc1_b)
    sv_h1, sv_init = {}, {}
    sv_h1['ACE'], sv_init['ACE'] = _feat([solvent_ACE], conv1_w, fc1_w, fc1_b)
    sv_h1['NMF'], sv_init['NMF'] = _feat([solvent_NMF], conv1_w, fc1_w, fc1_b)
    sv_h1['wat'], sv_init['wat'] = _feat([solvent_wat], conv1_w, fc1_w, fc1_b)

    b1_sv = jnp.tile(conv1_b, _B).reshape(1, -1)
    b2_sv = jnp.tile(conv2_b, _B).reshape(1, -1)
    b1_su = jnp.tile(conv1_b, 3 * _B).reshape(1, -1)
    b2_su = jnp.tile(conv2_b, 3 * _B).reshape(1, -1)
    w2_sv = jnp.kron(jnp.eye(_B, dtype=jnp.float32), conv2_w)
    w2_su = jnp.kron(jnp.eye(3 * _B, dtype=jnp.float32), conv2_w)

    p_su = _gcn_pool(solute_adj, su_h1, su_init, b1_su, b2_su, w2_su,
                     nmf_span=(_B * _NHID, 2 * _B * _NHID))
    p_ace = _gcn_pool(solvent_adj_ACE, sv_h1['ACE'], sv_init['ACE'],
                      b1_sv, b2_sv, w2_sv, nmf_span=None)
    p_nmf = _gcn_pool(solvent_adj_NMF, sv_h1['NMF'], sv_init['NMF'],
                      b1_sv, b2_sv, w2_sv, nmf_span=None)
    p_wat = _gcn_pool(solvent_adj_wat, sv_h1['wat'], sv_init['wat'],
                      b1_sv, b2_sv, w2_sv, nmf_span=None)

    c = _NCLASS
    rows = [
        jnp.concatenate([p_su[0 * c:1 * c], p_ace[0:c]]),
        jnp.concatenate([p_su[1 * c:2 * c], p_ace[c:2 * c]]),
        jnp.concatenate([p_su[2 * c:3 * c], p_nmf[0:c]]),
        jnp.concatenate([p_su[3 * c:4 * c], p_nmf[c:2 * c]]),
        jnp.concatenate([p_su[4 * c:5 * c], p_wat[0:c]]),
        jnp.concatenate([p_su[5 * c:6 * c], p_wat[c:2 * c]]),
    ]
    x8 = jnp.pad(jnp.stack(rows), ((0, 2), (0, 0)))
    out = _head(x8, fc2_w, fc2_b, fc3_w, fc3_b, fc4_w, fc4_b, fc5_w, fc5_b)
    return out[:6, :1]


# speedup vs baseline: 1.0638x; 1.0075x over previous
import functools

import jax
import jax.numpy as jnp
from jax.experimental import pallas as pl
from jax.experimental.pallas import tpu as pltpu

_NFEAT = 128
_NHID = 64
_NCLASS = 32
_B = 2

_ROW_BLK = 448
_L2_BLK = 448


def _feat_body(n_sets, w1_ref, wf_ref, bf_ref, *refs):
    x_refs = refs[:n_sets]
    h1_ref, init_ref = refs[n_sets], refs[n_sets + 1]
    w1 = w1_ref[...]
    wf = wf_ref[...]
    bf = bf_ref[...]
    h1_parts = []
    init_parts = []
    for x_ref in x_refs:
        for b in range(_B):
            xb = x_ref[b]
            h1_parts.append(
                jnp.dot(xb, w1, preferred_element_type=jnp.float32))
            init_parts.append(
                jnp.dot(xb, wf, preferred_element_type=jnp.float32) + bf)
    h1_ref[...] = jnp.concatenate(h1_parts, axis=1).astype(jnp.bfloat16)
    init_ref[...] = jnp.concatenate(init_parts, axis=1)


def _feat(xs, conv1_w, fc1_w, fc1_b):
    n_sets = len(xs)
    n = xs[0].shape[1]
    grid = pl.cdiv(n, _ROW_BLK)
    return pl.pallas_call(
        functools.partial(_feat_body, n_sets),
        grid=(grid,),
        in_specs=[
            pl.BlockSpec((_NFEAT, _NHID), lambda i: (0, 0)),
            pl.BlockSpec((_NFEAT, _NCLASS), lambda i: (0, 0)),
            pl.BlockSpec((1, _NCLASS), lambda i: (0, 0)),
        ] + [
            pl.BlockSpec((_B, _ROW_BLK, _NFEAT), lambda i: (0, i, 0))
            for _ in range(n_sets)
        ],
        out_specs=[
            pl.BlockSpec((_ROW_BLK, n_sets * _B * _NHID), lambda i: (i, 0)),
            pl.BlockSpec((_ROW_BLK, n_sets * _B * _NCLASS), lambda i: (i, 0)),
        ],
        out_shape=[
            jax.ShapeDtypeStruct((n, n_sets * _B * _NHID), jnp.bfloat16),
            jax.ShapeDtypeStruct((n, n_sets * _B * _NCLASS), jnp.float32),
        ],
    )(conv1_w, fc1_w, fc1_b.reshape(1, _NCLASS), *xs)


def _act(t, nmf_span):
    if nmf_span is None:
        return jnp.maximum(t, 0.0)
    lo, hi = nmf_span
    col = jax.lax.broadcasted_iota(jnp.int32, t.shape, 1)
    keep_linear = (col >= lo) & (col < hi)
    return jnp.where(keep_linear, t, jnp.maximum(t, 0.0))


def _layer1_body(nmf_span, adj_ref, h1_ref, b1_ref, w2_ref, h2_ref):
    s = _act(jnp.dot(adj_ref[...], h1_ref[...],
                     preferred_element_type=jnp.float32) + b1_ref[...],
             nmf_span)
    h2_ref[...] = jnp.dot(
        s, w2_ref[...], preferred_element_type=jnp.float32
    ).astype(jnp.bfloat16)


def _layer2_body(n_rows, adj_ref, h2_ref, init_ref, b2_ref, pool_ref):
    i = pl.program_id(0)
    t = (
        jnp.dot(adj_ref[...], h2_ref[...],
                preferred_element_type=jnp.float32)
        + b2_ref[...] + init_ref[...]
    )
    rows = jax.lax.broadcasted_iota(jnp.int32, t.shape, 0) + i * _L2_BLK
    t = jnp.where(rows < n_rows, t, -jnp.inf)
    m = jnp.max(t, axis=0, keepdims=True)
    m8 = jnp.broadcast_to(m, (8, t.shape[1]))

    @pl.when(i == 0)
    def _():
        pool_ref[...] = m8

    @pl.when(i > 0)
    def _():
        pool_ref[...] = jnp.maximum(pool_ref[...], m8)


def _gcn_pool(adj, h1, init, b1, b2, w2d, nmf_span):
    n = adj.shape[0]
    f1 = h1.shape[1]
    f2 = w2d.shape[1]
    params = pltpu.CompilerParams(vmem_limit_bytes=64 * 1024 * 1024)
    h2 = pl.pallas_call(
        functools.partial(_layer1_body, nmf_span),
        grid=(pl.cdiv(n, _ROW_BLK),),
        in_specs=[
            pl.BlockSpec((_ROW_BLK, n), lambda i: (i, 0)),
            pl.BlockSpec((n, f1), lambda i: (0, 0)),
            pl.BlockSpec((1, f1), lambda i: (0, 0)),
            pl.BlockSpec((f1, f2), lambda i: (0, 0)),
        ],
        out_specs=pl.BlockSpec((_ROW_BLK, f2), lambda i: (i, 0)),
        out_shape=jax.ShapeDtypeStruct((n, f2), jnp.bfloat16),
        compiler_params=params,
    )(adj, h1, b1, w2d)
    pooled = pl.pallas_call(
        functools.partial(_layer2_body, n),
        grid=(pl.cdiv(n, _L2_BLK),),
        in_specs=[
            pl.BlockSpec((_L2_BLK, n), lambda i: (i, 0)),
            pl.BlockSpec((n, f2), lambda i: (0, 0)),
            pl.BlockSpec((_L2_BLK, f2), lambda i: (i, 0)),
            pl.BlockSpec((1, f2), lambda i: (0, 0)),
        ],
        out_specs=pl.BlockSpec((8, f2), lambda i: (0, 0)),
        out_shape=jax.ShapeDtypeStruct((8, f2), jnp.float32),
        compiler_params=params,
    )(adj, h2, init, b2)
    return pooled[0]


def _head_body(x_ref, w2_ref, b2_ref, w3_ref, b3_ref, w4_ref, b4_ref,
               w5_ref, b5_ref, out_ref):
    x = x_ref[...]
    x = jnp.maximum(jnp.dot(x, w2_ref[...], preferred_element_type=jnp.float32)
                    + b2_ref[...], 0.0)
    x = jnp.maximum(jnp.dot(x, w3_ref[...], preferred_element_type=jnp.float32)
                    + b3_ref[...], 0.0)
    x = jnp.maximum(jnp.dot(x, w4_ref[...], preferred_element_type=jnp.float32)
                    + b4_ref[...], 0.0)
    d = jnp.dot(x, w5_ref[...], preferred_element_type=jnp.float32) + b5_ref[...]
    out_ref[...] = d + jnp.zeros((8, 8), jnp.float32)


def _head(x8, fc2_w, fc2_b, fc3_w, fc3_b, fc4_w, fc4_b, fc5_w, fc5_b):
    full = lambda shape: pl.BlockSpec(shape, lambda: (0,) * len(shape))
    return pl.pallas_call(
        _head_body,
        in_specs=[
            full((8, 2 * _NCLASS)),
            full(fc2_w.shape), full((1, _NCLASS)),
            full(fc3_w.shape), full((1, 64)),
            full(fc4_w.shape), full((1, 32)),
            full(fc5_w.shape), full((1, 1)),
        ],
        out_specs=full((8, 8)),
        out_shape=jax.ShapeDtypeStruct((8, 8), jnp.float32),
    )(x8, fc2_w, fc2_b.reshape(1, -1), fc3_w, fc3_b.reshape(1, -1),
      fc4_w, fc4_b.reshape(1, -1), fc5_w, fc5_b.reshape(1, -1))


def kernel(solute_ACE, solvent_ACE, solute_adj, solvent_adj_ACE, solute_NMF,
           solvent_NMF, solvent_adj_NMF, solute_wat, solvent_wat,
           solvent_adj_wat, fc1_w, fc1_b, conv1_w, conv1_b, conv2_w, conv2_b,
           fc2_w, fc2_b, fc3_w, fc3_b, fc4_w, fc4_b, fc5_w, fc5_b):
    su_h1, su_init = _feat([solute_ACE, solute_NMF, solute_wat],
                           conv1_w, fc1_w, fc1_b)
    sv_h1, sv_init = {}, {}
    sv_h1['ACE'], sv_init['ACE'] = _feat([solvent_ACE], conv1_w, fc1_w, fc1_b)
    sv_h1['NMF'], sv_init['NMF'] = _feat([solvent_NMF], conv1_w, fc1_w, fc1_b)
    sv_h1['wat'], sv_init['wat'] = _feat([solvent_wat], conv1_w, fc1_w, fc1_b)

    b1_sv = jnp.tile(conv1_b, _B).reshape(1, -1)
    b2_sv = jnp.tile(conv2_b, _B).reshape(1, -1)
    b1_su = jnp.tile(conv1_b, 3 * _B).reshape(1, -1)
    b2_su = jnp.tile(conv2_b, 3 * _B).reshape(1, -1)
    w2_sv = jnp.kron(jnp.eye(_B, dtype=jnp.float32), conv2_w)
    w2_su = jnp.kron(jnp.eye(3 * _B, dtype=jnp.float32), conv2_w)

    p_su = _gcn_pool(solute_adj, su_h1, su_init, b1_su, b2_su, w2_su,
                     nmf_span=(_B * _NHID, 2 * _B * _NHID))
    p_ace = _gcn_pool(solvent_adj_ACE, sv_h1['ACE'], sv_init['ACE'],
                      b1_sv, b2_sv, w2_sv, nmf_span=None)
    p_nmf = _gcn_pool(solvent_adj_NMF, sv_h1['NMF'], sv_init['NMF'],
                      b1_sv, b2_sv, w2_sv, nmf_span=None)
    p_wat = _gcn_pool(solvent_adj_wat, sv_h1['wat'], sv_init['wat'],
                      b1_sv, b2_sv, w2_sv, nmf_span=None)

    c = _NCLASS
    rows = [
        jnp.concatenate([p_su[0 * c:1 * c], p_ace[0:c]]),
        jnp.concatenate([p_su[1 * c:2 * c], p_ace[c:2 * c]]),
        jnp.concatenate([p_su[2 * c:3 * c], p_nmf[0:c]]),
        jnp.concatenate([p_su[3 * c:4 * c], p_nmf[c:2 * c]]),
        jnp.concatenate([p_su[4 * c:5 * c], p_wat[0:c]]),
        jnp.concatenate([p_su[5 * c:6 * c], p_wat[c:2 * c]]),
    ]
    x8 = jnp.pad(jnp.stack(rows), ((0, 2), (0, 0)))
    out = _head(x8, fc2_w, fc2_b, fc3_w, fc3_b, fc4_w, fc4_b, fc5_w, fc5_b)
    return out[:6, :1]


# speedup vs baseline: 1.0686x; 1.0045x over previous
import functools

import jax
import jax.numpy as jnp
from jax.experimental import pallas as pl
from jax.experimental.pallas import tpu as pltpu

_NFEAT = 128
_NHID = 64
_NCLASS = 32
_B = 2

_ROW_BLK = 448
_L2_BLK = 448


def _feat_body(n_sets, w1_ref, wf_ref, bf_ref, *refs):
    x_refs = refs[:n_sets]
    h1_ref, init_ref = refs[n_sets], refs[n_sets + 1]
    w1 = w1_ref[...]
    wf = wf_ref[...]
    bf = bf_ref[...]
    h1_parts = []
    init_parts = []
    for x_ref in x_refs:
        for b in range(_B):
            xb = x_ref[b]
            h1_parts.append(
                jnp.dot(xb, w1, preferred_element_type=jnp.float32))
            init_parts.append(
                jnp.dot(xb, wf, preferred_element_type=jnp.float32) + bf)
    h1_ref[...] = jnp.concatenate(h1_parts, axis=1).astype(jnp.bfloat16)
    init_ref[...] = jnp.concatenate(init_parts, axis=1)


def _feat(xs, conv1_w, fc1_w, fc1_b):
    n_sets = len(xs)
    n = xs[0].shape[1]
    grid = pl.cdiv(n, _ROW_BLK)
    return pl.pallas_call(
        functools.partial(_feat_body, n_sets),
        grid=(grid,),
        in_specs=[
            pl.BlockSpec((_NFEAT, _NHID), lambda i: (0, 0)),
            pl.BlockSpec((_NFEAT, _NCLASS), lambda i: (0, 0)),
            pl.BlockSpec((1, _NCLASS), lambda i: (0, 0)),
        ] + [
            pl.BlockSpec((_B, _ROW_BLK, _NFEAT), lambda i: (0, i, 0))
            for _ in range(n_sets)
        ],
        out_specs=[
            pl.BlockSpec((_ROW_BLK, n_sets * _B * _NHID), lambda i: (i, 0)),
            pl.BlockSpec((_ROW_BLK, n_sets * _B * _NCLASS), lambda i: (i, 0)),
        ],
        out_shape=[
            jax.ShapeDtypeStruct((n, n_sets * _B * _NHID), jnp.bfloat16),
            jax.ShapeDtypeStruct((n, n_sets * _B * _NCLASS), jnp.float32),
        ],
    )(conv1_w, fc1_w, fc1_b.reshape(1, _NCLASS), *xs)


def _act(t, nmf_span):
    if nmf_span is None:
        return jnp.maximum(t, 0.0)
    lo, hi = nmf_span
    col = jax.lax.broadcasted_iota(jnp.int32, t.shape, 1)
    keep_linear = (col >= lo) & (col < hi)
    return jnp.where(keep_linear, t, jnp.maximum(t, 0.0))


def _layer1_body(nmf_span, adj_ref, h1_ref, b1_ref, w2_ref, h2_ref):
    s = _act(jnp.dot(adj_ref[...], h1_ref[...],
                     preferred_element_type=jnp.float32) + b1_ref[...],
             nmf_span)
    h2_ref[...] = jnp.dot(
        s, w2_ref[...], preferred_element_type=jnp.float32
    ).astype(jnp.bfloat16)


def _layer2_body(n_rows, adj_ref, h2_ref, init_ref, b2_ref, pool_ref):
    i = pl.program_id(0)
    t = (
        jnp.dot(adj_ref[...], h2_ref[...],
                preferred_element_type=jnp.float32)
        + b2_ref[...] + init_ref[...]
    )
    rows = jax.lax.broadcasted_iota(jnp.int32, t.shape, 0) + i * _L2_BLK
    t = jnp.where(rows < n_rows, t, -jnp.inf)
    m = jnp.max(t, axis=0, keepdims=True)
    m8 = jnp.broadcast_to(m, (8, t.shape[1]))

    @pl.when(i == 0)
    def _():
        pool_ref[...] = m8

    @pl.when(i > 0)
    def _():
        pool_ref[...] = jnp.maximum(pool_ref[...], m8)


def _gcn_pool(adj, h1, init, b1, b2, w2d, nmf_span):
    n = adj.shape[0]
    f1 = h1.shape[1]
    f2 = w2d.shape[1]
    params = pltpu.CompilerParams(vmem_limit_bytes=64 * 1024 * 1024)
    h2 = pl.pallas_call(
        functools.partial(_layer1_body, nmf_span),
        grid=(pl.cdiv(n, _ROW_BLK),),
        in_specs=[
            pl.BlockSpec((_ROW_BLK, n), lambda i: (i, 0)),
            pl.BlockSpec((n, f1), lambda i: (0, 0)),
            pl.BlockSpec((1, f1), lambda i: (0, 0)),
            pl.BlockSpec((f1, f2), lambda i: (0, 0)),
        ],
        out_specs=pl.BlockSpec((_ROW_BLK, f2), lambda i: (i, 0)),
        out_shape=jax.ShapeDtypeStruct((n, f2), jnp.bfloat16),
        compiler_params=params,
    )(adj, h1, b1, w2d)
    pooled = pl.pallas_call(
        functools.partial(_layer2_body, n),
        grid=(pl.cdiv(n, _L2_BLK),),
        in_specs=[
            pl.BlockSpec((_L2_BLK, n), lambda i: (i, 0)),
            pl.BlockSpec((n, f2), lambda i: (0, 0)),
            pl.BlockSpec((_L2_BLK, f2), lambda i: (i, 0)),
            pl.BlockSpec((1, f2), lambda i: (0, 0)),
        ],
        out_specs=pl.BlockSpec((8, f2), lambda i: (0, 0)),
        out_shape=jax.ShapeDtypeStruct((8, f2), jnp.float32),
        compiler_params=params,
    )(adj, h2, init, b2)
    return pooled[0]


def _solute_body(nmf_span, adj_ref, xa_ref, xn_ref, xw_ref, w1_ref, wf_ref,
                 bf_ref, b1_ref, b2_ref, w2d_ref, pool_ref):
    adj = adj_ref[...]
    w1 = w1_ref[...]
    wf = wf_ref[...]
    bf = bf_ref[...]
    h1_parts = []
    init_parts = []
    for x_ref in (xa_ref, xn_ref, xw_ref):
        for b in range(_B):
            xb = x_ref[b]
            h1_parts.append(
                jnp.dot(xb, w1, preferred_element_type=jnp.float32))
            init_parts.append(
                jnp.dot(xb, wf, preferred_element_type=jnp.float32) + bf)
    h1 = jnp.concatenate(h1_parts, axis=1)
    init = jnp.concatenate(init_parts, axis=1)
    s = _act(jnp.dot(adj, h1, preferred_element_type=jnp.float32)
             + b1_ref[...], nmf_span)
    h2 = jnp.dot(s, w2d_ref[...], preferred_element_type=jnp.float32)
    t = jnp.dot(adj, h2, preferred_element_type=jnp.float32) \
        + b2_ref[...] + init
    m = jnp.max(t, axis=0, keepdims=True)
    pool_ref[...] = jnp.broadcast_to(m, pool_ref.shape)


def _solute_pool(adj, xs, conv1_w, fc1_w, fc1_b, b1, b2, w2d, nmf_span):
    n = adj.shape[0]
    f1 = w2d.shape[0]
    f2 = w2d.shape[1]
    full = lambda shape: pl.BlockSpec(shape, lambda: (0,) * len(shape))
    pooled = pl.pallas_call(
        functools.partial(_solute_body, nmf_span),
        in_specs=[
            full((n, n)),
            full((_B, n, _NFEAT)), full((_B, n, _NFEAT)),
            full((_B, n, _NFEAT)),
            full((_NFEAT, _NHID)), full((_NFEAT, _NCLASS)),
            full((1, _NCLASS)), full((1, f1)), full((1, f2)),
            full((f1, f2)),
        ],
        out_specs=full((8, f2)),
        out_shape=jax.ShapeDtypeStruct((8, f2), jnp.float32),
        compiler_params=pltpu.CompilerParams(
            vmem_limit_bytes=64 * 1024 * 1024),
    )(adj, xs[0], xs[1], xs[2], conv1_w, fc1_w,
      fc1_b.reshape(1, _NCLASS), b1, b2, w2d)
    return pooled[0]


def _head_body(x_ref, w2_ref, b2_ref, w3_ref, b3_ref, w4_ref, b4_ref,
               w5_ref, b5_ref, out_ref):
    x = x_ref[...]
    x = jnp.maximum(jnp.dot(x, w2_ref[...], preferred_element_type=jnp.float32)
                    + b2_ref[...], 0.0)
    x = jnp.maximum(jnp.dot(x, w3_ref[...], preferred_element_type=jnp.float32)
                    + b3_ref[...], 0.0)
    x = jnp.maximum(jnp.dot(x, w4_ref[...], preferred_element_type=jnp.float32)
                    + b4_ref[...], 0.0)
    d = jnp.dot(x, w5_ref[...], preferred_element_type=jnp.float32) + b5_ref[...]
    out_ref[...] = d + jnp.zeros((8, 8), jnp.float32)


def _head(x8, fc2_w, fc2_b, fc3_w, fc3_b, fc4_w, fc4_b, fc5_w, fc5_b):
    full = lambda shape: pl.BlockSpec(shape, lambda: (0,) * len(shape))
    return pl.pallas_call(
        _head_body,
        in_specs=[
            full((8, 2 * _NCLASS)),
            full(fc2_w.shape), full((1, _NCLASS)),
            full(fc3_w.shape), full((1, 64)),
            full(fc4_w.shape), full((1, 32)),
            full(fc5_w.shape), full((1, 1)),
        ],
        out_specs=full((8, 8)),
        out_shape=jax.ShapeDtypeStruct((8, 8), jnp.float32),
    )(x8, fc2_w, fc2_b.reshape(1, -1), fc3_w, fc3_b.reshape(1, -1),
      fc4_w, fc4_b.reshape(1, -1), fc5_w, fc5_b.reshape(1, -1))


def kernel(solute_ACE, solvent_ACE, solute_adj, solvent_adj_ACE, solute_NMF,
           solvent_NMF, solvent_adj_NMF, solute_wat, solvent_wat,
           solvent_adj_wat, fc1_w, fc1_b, conv1_w, conv1_b, conv2_w, conv2_b,
           fc2_w, fc2_b, fc3_w, fc3_b, fc4_w, fc4_b, fc5_w, fc5_b):
    sv_h1, sv_init = {}, {}
    sv_h1['ACE'], sv_init['ACE'] = _feat([solvent_ACE], conv1_w, fc1_w, fc1_b)
    sv_h1['NMF'], sv_init['NMF'] = _feat([solvent_NMF], conv1_w, fc1_w, fc1_b)
    sv_h1['wat'], sv_init['wat'] = _feat([solvent_wat], conv1_w, fc1_w, fc1_b)

    b1_sv = jnp.tile(conv1_b, _B).reshape(1, -1)
    b2_sv = jnp.tile(conv2_b, _B).reshape(1, -1)
    b1_su = jnp.tile(conv1_b, 3 * _B).reshape(1, -1)
    b2_su = jnp.tile(conv2_b, 3 * _B).reshape(1, -1)
    w2_sv = jnp.kron(jnp.eye(_B, dtype=jnp.float32), conv2_w)
    w2_su = jnp.kron(jnp.eye(3 * _B, dtype=jnp.float32), conv2_w)

    p_su = _solute_pool(solute_adj, [solute_ACE, solute_NMF, solute_wat],
                        conv1_w, fc1_w, fc1_b, b1_su, b2_su, w2_su,
                        nmf_span=(_B * _NHID, 2 * _B * _NHID))
    p_ace = _gcn_pool(solvent_adj_ACE, sv_h1['ACE'], sv_init['ACE'],
                      b1_sv, b2_sv, w2_sv, nmf_span=None)
    p_nmf = _gcn_pool(solvent_adj_NMF, sv_h1['NMF'], sv_init['NMF'],
                      b1_sv, b2_sv, w2_sv, nmf_span=None)
    p_wat = _gcn_pool(solvent_adj_wat, sv_h1['wat'], sv_init['wat'],
                      b1_sv, b2_sv, w2_sv, nmf_span=None)

    c = _NCLASS
    rows = [
        jnp.concatenate([p_su[0 * c:1 * c], p_ace[0:c]]),
        jnp.concatenate([p_su[1 * c:2 * c], p_ace[c:2 * c]]),
        jnp.concatenate([p_su[2 * c:3 * c], p_nmf[0:c]]),
        jnp.concatenate([p_su[3 * c:4 * c], p_nmf[c:2 * c]]),
        jnp.concatenate([p_su[4 * c:5 * c], p_wat[0:c]]),
        jnp.concatenate([p_su[5 * c:6 * c], p_wat[c:2 * c]]),
    ]
    x8 = jnp.pad(jnp.stack(rows), ((0, 2), (0, 0)))
    out = _head(x8, fc2_w, fc2_b, fc3_w, fc3_b, fc4_w, fc4_b, fc5_w, fc5_b)
    return out[:6, :1]


# speedup vs baseline: 1.0714x; 1.0027x over previous
import functools

import jax
import jax.numpy as jnp
from jax.experimental import pallas as pl
from jax.experimental.pallas import tpu as pltpu

_NFEAT = 128
_NHID = 64
_NCLASS = 32
_B = 2

_ROW_BLK = 480
_L2_BLK = 480


def _feat_body(n_sets, w1_ref, wf_ref, bf_ref, *refs):
    x_refs = refs[:n_sets]
    h1_ref, init_ref = refs[n_sets], refs[n_sets + 1]
    w1 = w1_ref[...]
    wf = wf_ref[...]
    bf = bf_ref[...]
    h1_parts = []
    init_parts = []
    for x_ref in x_refs:
        for b in range(_B):
            xb = x_ref[b]
            h1_parts.append(
                jnp.dot(xb, w1, preferred_element_type=jnp.float32))
            init_parts.append(
                jnp.dot(xb, wf, preferred_element_type=jnp.float32) + bf)
    h1_ref[...] = jnp.concatenate(h1_parts, axis=1).astype(jnp.bfloat16)
    init_ref[...] = jnp.concatenate(init_parts, axis=1)


def _feat(xs, conv1_w, fc1_w, fc1_b):
    n_sets = len(xs)
    n = xs[0].shape[1]
    grid = pl.cdiv(n, _ROW_BLK)
    return pl.pallas_call(
        functools.partial(_feat_body, n_sets),
        grid=(grid,),
        in_specs=[
            pl.BlockSpec((_NFEAT, _NHID), lambda i: (0, 0)),
            pl.BlockSpec((_NFEAT, _NCLASS), lambda i: (0, 0)),
            pl.BlockSpec((1, _NCLASS), lambda i: (0, 0)),
        ] + [
            pl.BlockSpec((_B, _ROW_BLK, _NFEAT), lambda i: (0, i, 0))
            for _ in range(n_sets)
        ],
        out_specs=[
            pl.BlockSpec((_ROW_BLK, n_sets * _B * _NHID), lambda i: (i, 0)),
            pl.BlockSpec((_ROW_BLK, n_sets * _B * _NCLASS), lambda i: (i, 0)),
        ],
        out_shape=[
            jax.ShapeDtypeStruct((n, n_sets * _B * _NHID), jnp.bfloat16),
            jax.ShapeDtypeStruct((n, n_sets * _B * _NCLASS), jnp.float32),
        ],
    )(conv1_w, fc1_w, fc1_b.reshape(1, _NCLASS), *xs)


def _act(t, nmf_span):
    if nmf_span is None:
        return jnp.maximum(t, 0.0)
    lo, hi = nmf_span
    col = jax.lax.broadcasted_iota(jnp.int32, t.shape, 1)
    keep_linear = (col >= lo) & (col < hi)
    return jnp.where(keep_linear, t, jnp.maximum(t, 0.0))


def _layer1_body(nmf_span, adj_ref, h1_ref, b1_ref, w2_ref, h2_ref):
    s = _act(jnp.dot(adj_ref[...], h1_ref[...],
                     preferred_element_type=jnp.float32) + b1_ref[...],
             nmf_span)
    h2_ref[...] = jnp.dot(
        s, w2_ref[...], preferred_element_type=jnp.float32
    ).astype(jnp.bfloat16)


def _layer2_body(n_rows, adj_ref, h2_ref, init_ref, b2_ref, pool_ref):
    i = pl.program_id(0)
    t = (
        jnp.dot(adj_ref[...], h2_ref[...],
                preferred_element_type=jnp.float32)
        + b2_ref[...] + init_ref[...]
    )
    rows = jax.lax.broadcasted_iota(jnp.int32, t.shape, 0) + i * _L2_BLK
    t = jnp.where(rows < n_rows, t, -jnp.inf)
    m = jnp.max(t, axis=0, keepdims=True)
    m8 = jnp.broadcast_to(m, (8, t.shape[1]))

    @pl.when(i == 0)
    def _():
        pool_ref[...] = m8

    @pl.when(i > 0)
    def _():
        pool_ref[...] = jnp.maximum(pool_ref[...], m8)


def _gcn_pool(adj, h1, init, b1, b2, w2d, nmf_span):
    n = adj.shape[0]
    f1 = h1.shape[1]
    f2 = w2d.shape[1]
    params = pltpu.CompilerParams(vmem_limit_bytes=64 * 1024 * 1024)
    h2 = pl.pallas_call(
        functools.partial(_layer1_body, nmf_span),
        grid=(pl.cdiv(n, _ROW_BLK),),
        in_specs=[
            pl.BlockSpec((_ROW_BLK, n), lambda i: (i, 0)),
            pl.BlockSpec((n, f1), lambda i: (0, 0)),
            pl.BlockSpec((1, f1), lambda i: (0, 0)),
            pl.BlockSpec((f1, f2), lambda i: (0, 0)),
        ],
        out_specs=pl.BlockSpec((_ROW_BLK, f2), lambda i: (i, 0)),
        out_shape=jax.ShapeDtypeStruct((n, f2), jnp.bfloat16),
        compiler_params=params,
    )(adj, h1, b1, w2d)
    pooled = pl.pallas_call(
        functools.partial(_layer2_body, n),
        grid=(pl.cdiv(n, _L2_BLK),),
        in_specs=[
            pl.BlockSpec((_L2_BLK, n), lambda i: (i, 0)),
            pl.BlockSpec((n, f2), lambda i: (0, 0)),
            pl.BlockSpec((_L2_BLK, f2), lambda i: (i, 0)),
            pl.BlockSpec((1, f2), lambda i: (0, 0)),
        ],
        out_specs=pl.BlockSpec((8, f2), lambda i: (0, 0)),
        out_shape=jax.ShapeDtypeStruct((8, f2), jnp.float32),
        compiler_params=params,
    )(adj, h2, init, b2)
    return pooled[0]


def _solute_body(nmf_span, adj_ref, xa_ref, xn_ref, xw_ref, w1_ref, wf_ref,
                 bf_ref, b1_ref, b2_ref, w2d_ref, pool_ref):
    adj = adj_ref[...]
    w1 = w1_ref[...]
    wf = wf_ref[...]
    bf = bf_ref[...]
    h1_parts = []
    init_parts = []
    for x_ref in (xa_ref, xn_ref, xw_ref):
        for b in range(_B):
            xb = x_ref[b]
            h1_parts.append(
                jnp.dot(xb, w1, preferred_element_type=jnp.float32))
            init_parts.append(
                jnp.dot(xb, wf, preferred_element_type=jnp.float32) + bf)
    h1 = jnp.concatenate(h1_parts, axis=1)
    init = jnp.concatenate(init_parts, axis=1)
    s = _act(jnp.dot(adj, h1, preferred_element_type=jnp.float32)
             + b1_ref[...], nmf_span)
    h2 = jnp.dot(s, w2d_ref[...], preferred_element_type=jnp.float32)
    t = jnp.dot(adj, h2, preferred_element_type=jnp.float32) \
        + b2_ref[...] + init
    m = jnp.max(t, axis=0, keepdims=True)
    pool_ref[...] = jnp.broadcast_to(m, pool_ref.shape)


def _solute_pool(adj, xs, conv1_w, fc1_w, fc1_b, b1, b2, w2d, nmf_span):
    n = adj.shape[0]
    f1 = w2d.shape[0]
    f2 = w2d.shape[1]
    full = lambda shape: pl.BlockSpec(shape, lambda: (0,) * len(shape))
    pooled = pl.pallas_call(
        functools.partial(_solute_body, nmf_span),
        in_specs=[
            full((n, n)),
            full((_B, n, _NFEAT)), full((_B, n, _NFEAT)),
            full((_B, n, _NFEAT)),
            full((_NFEAT, _NHID)), full((_NFEAT, _NCLASS)),
            full((1, _NCLASS)), full((1, f1)), full((1, f2)),
            full((f1, f2)),
        ],
        out_specs=full((8, f2)),
        out_shape=jax.ShapeDtypeStruct((8, f2), jnp.float32),
        compiler_params=pltpu.CompilerParams(
            vmem_limit_bytes=64 * 1024 * 1024),
    )(adj, xs[0], xs[1], xs[2], conv1_w, fc1_w,
      fc1_b.reshape(1, _NCLASS), b1, b2, w2d)
    return pooled[0]


def _head_body(x_ref, w2_ref, b2_ref, w3_ref, b3_ref, w4_ref, b4_ref,
               w5_ref, b5_ref, out_ref):
    x = x_ref[...]
    x = jnp.maximum(jnp.dot(x, w2_ref[...], preferred_element_type=jnp.float32)
                    + b2_ref[...], 0.0)
    x = jnp.maximum(jnp.dot(x, w3_ref[...], preferred_element_type=jnp.float32)
                    + b3_ref[...], 0.0)
    x = jnp.maximum(jnp.dot(x, w4_ref[...], preferred_element_type=jnp.float32)
                    + b4_ref[...], 0.0)
    d = jnp.dot(x, w5_ref[...], preferred_element_type=jnp.float32) + b5_ref[...]
    out_ref[...] = d + jnp.zeros((8, 8), jnp.float32)


def _head(x8, fc2_w, fc2_b, fc3_w, fc3_b, fc4_w, fc4_b, fc5_w, fc5_b):
    full = lambda shape: pl.BlockSpec(shape, lambda: (0,) * len(shape))
    return pl.pallas_call(
        _head_body,
        in_specs=[
            full((8, 2 * _NCLASS)),
            full(fc2_w.shape), full((1, _NCLASS)),
            full(fc3_w.shape), full((1, 64)),
            full(fc4_w.shape), full((1, 32)),
            full(fc5_w.shape), full((1, 1)),
        ],
        out_specs=full((8, 8)),
        out_shape=jax.ShapeDtypeStruct((8, 8), jnp.float32),
    )(x8, fc2_w, fc2_b.reshape(1, -1), fc3_w, fc3_b.reshape(1, -1),
      fc4_w, fc4_b.reshape(1, -1), fc5_w, fc5_b.reshape(1, -1))


def kernel(solute_ACE, solvent_ACE, solute_adj, solvent_adj_ACE, solute_NMF,
           solvent_NMF, solvent_adj_NMF, solute_wat, solvent_wat,
           solvent_adj_wat, fc1_w, fc1_b, conv1_w, conv1_b, conv2_w, conv2_b,
           fc2_w, fc2_b, fc3_w, fc3_b, fc4_w, fc4_b, fc5_w, fc5_b):
    sv_h1, sv_init = {}, {}
    sv_h1['ACE'], sv_init['ACE'] = _feat([solvent_ACE], conv1_w, fc1_w, fc1_b)
    sv_h1['NMF'], sv_init['NMF'] = _feat([solvent_NMF], conv1_w, fc1_w, fc1_b)
    sv_h1['wat'], sv_init['wat'] = _feat([solvent_wat], conv1_w, fc1_w, fc1_b)

    b1_sv = jnp.tile(conv1_b, _B).reshape(1, -1)
    b2_sv = jnp.tile(conv2_b, _B).reshape(1, -1)
    b1_su = jnp.tile(conv1_b, 3 * _B).reshape(1, -1)
    b2_su = jnp.tile(conv2_b, 3 * _B).reshape(1, -1)
    w2_sv = jnp.kron(jnp.eye(_B, dtype=jnp.float32), conv2_w)
    w2_su = jnp.kron(jnp.eye(3 * _B, dtype=jnp.float32), conv2_w)

    p_su = _solute_pool(solute_adj, [solute_ACE, solute_NMF, solute_wat],
                        conv1_w, fc1_w, fc1_b, b1_su, b2_su, w2_su,
                        nmf_span=(_B * _NHID, 2 * _B * _NHID))
    p_ace = _gcn_pool(solvent_adj_ACE, sv_h1['ACE'], sv_init['ACE'],
                      b1_sv, b2_sv, w2_sv, nmf_span=None)
    p_nmf = _gcn_pool(solvent_adj_NMF, sv_h1['NMF'], sv_init['NMF'],
                      b1_sv, b2_sv, w2_sv, nmf_span=None)
    p_wat = _gcn_pool(solvent_adj_wat, sv_h1['wat'], sv_init['wat'],
                      b1_sv, b2_sv, w2_sv, nmf_span=None)

    c = _NCLASS
    rows = [
        jnp.concatenate([p_su[0 * c:1 * c], p_ace[0:c]]),
        jnp.concatenate([p_su[1 * c:2 * c], p_ace[c:2 * c]]),
        jnp.concatenate([p_su[2 * c:3 * c], p_nmf[0:c]]),
        jnp.concatenate([p_su[3 * c:4 * c], p_nmf[c:2 * c]]),
        jnp.concatenate([p_su[4 * c:5 * c], p_wat[0:c]]),
        jnp.concatenate([p_su[5 * c:6 * c], p_wat[c:2 * c]]),
    ]
    x8 = jnp.pad(jnp.stack(rows), ((0, 2), (0, 0)))
    out = _head(x8, fc2_w, fc2_b, fc3_w, fc3_b, fc4_w, fc4_b, fc5_w, fc5_b)
    return out[:6, :1]


# speedup vs baseline: 1.0807x; 1.0086x over previous
import functools

import jax
import jax.numpy as jnp
from jax.experimental import pallas as pl
from jax.experimental.pallas import tpu as pltpu

_NFEAT = 128
_NHID = 64
_NCLASS = 32
_B = 2

_ROW_BLK = 448


def _feat_body(n_sets, w1_ref, wf_ref, bf_ref, *refs):
    x_refs = refs[:n_sets]
    h1_ref, init_ref = refs[n_sets], refs[n_sets + 1]
    w1 = w1_ref[...]
    wf = wf_ref[...]
    bf = bf_ref[...]
    h1_parts = []
    init_parts = []
    for x_ref in x_refs:
        for b in range(_B):
            xb = x_ref[b]
            h1_parts.append(
                jnp.dot(xb, w1, preferred_element_type=jnp.float32))
            init_parts.append(
                jnp.dot(xb, wf, preferred_element_type=jnp.float32) + bf)
    h1_ref[...] = jnp.concatenate(h1_parts, axis=1).astype(jnp.bfloat16)
    init_ref[...] = jnp.concatenate(init_parts, axis=1)


def _feat(xs, conv1_w, fc1_w, fc1_b):
    n_sets = len(xs)
    n = xs[0].shape[1]
    grid = pl.cdiv(n, _ROW_BLK)
    return pl.pallas_call(
        functools.partial(_feat_body, n_sets),
        grid=(grid,),
        in_specs=[
            pl.BlockSpec((_NFEAT, _NHID), lambda i: (0, 0)),
            pl.BlockSpec((_NFEAT, _NCLASS), lambda i: (0, 0)),
            pl.BlockSpec((1, _NCLASS), lambda i: (0, 0)),
        ] + [
            pl.BlockSpec((_B, _ROW_BLK, _NFEAT), lambda i: (0, i, 0))
            for _ in range(n_sets)
        ],
        out_specs=[
            pl.BlockSpec((_ROW_BLK, n_sets * _B * _NHID), lambda i: (i, 0)),
            pl.BlockSpec((_ROW_BLK, n_sets * _B * _NCLASS), lambda i: (i, 0)),
        ],
        out_shape=[
            jax.ShapeDtypeStruct((n, n_sets * _B * _NHID), jnp.bfloat16),
            jax.ShapeDtypeStruct((n, n_sets * _B * _NCLASS), jnp.float32),
        ],
    )(conv1_w, fc1_w, fc1_b.reshape(1, _NCLASS), *xs)


def _act(t, nmf_span):
    if nmf_span is None:
        return jnp.maximum(t, 0.0)
    lo, hi = nmf_span
    col = jax.lax.broadcasted_iota(jnp.int32, t.shape, 1)
    keep_linear = (col >= lo) & (col < hi)
    return jnp.where(keep_linear, t, jnp.maximum(t, 0.0))


def _graph_body(n_rows, g1, adj_ref, h1_ref, b1_ref, w2_ref, init_ref,
                b2_ref, pool_ref, h2_scr):
    i = pl.program_id(0)

    @pl.when(i < g1)
    def _():
        s = _act(jnp.dot(adj_ref[...], h1_ref[...],
                         preferred_element_type=jnp.float32) + b1_ref[...],
                 None)
        h2_scr[pl.ds(i * _ROW_BLK, _ROW_BLK), :] = jnp.dot(
            s, w2_ref[...], preferred_element_type=jnp.float32
        ).astype(jnp.bfloat16)

    @pl.when(i >= g1)
    def _():
        k = i - g1
        h2 = h2_scr[...][0:n_rows, :]
        t = (
            jnp.dot(adj_ref[...], h2, preferred_element_type=jnp.float32)
            + b2_ref[...] + init_ref[...]
        )
        rows = jax.lax.broadcasted_iota(jnp.int32, t.shape, 0) + k * _ROW_BLK
        t = jnp.where(rows < n_rows, t, -jnp.inf)
        m8 = jnp.broadcast_to(jnp.max(t, axis=0, keepdims=True),
                              (8, t.shape[1]))

        @pl.when(k == 0)
        def _():
            pool_ref[...] = m8

        @pl.when(k > 0)
        def _():
            pool_ref[...] = jnp.maximum(pool_ref[...], m8)


def _gcn_pool(adj, h1, init, b1, b2, w2d, nmf_span):
    del nmf_span
    n = adj.shape[0]
    f1 = h1.shape[1]
    f2 = w2d.shape[1]
    g1 = pl.cdiv(n, _ROW_BLK)
    params = pltpu.CompilerParams(vmem_limit_bytes=64 * 1024 * 1024)
    phase_blk = lambda i: (jnp.where(i < g1, i, i - g1), 0)
    pooled = pl.pallas_call(
        functools.partial(_graph_body, n, g1),
        grid=(2 * g1,),
        in_specs=[
            pl.BlockSpec((_ROW_BLK, n), phase_blk),
            pl.BlockSpec((n, f1), lambda i: (0, 0)),
            pl.BlockSpec((1, f1), lambda i: (0, 0)),
            pl.BlockSpec((f1, f2), lambda i: (0, 0)),
            pl.BlockSpec((_ROW_BLK, f2), phase_blk),
            pl.BlockSpec((1, f2), lambda i: (0, 0)),
        ],
        out_specs=pl.BlockSpec((8, f2), lambda i: (0, 0)),
        out_shape=jax.ShapeDtypeStruct((8, f2), jnp.float32),
        scratch_shapes=[pltpu.VMEM((g1 * _ROW_BLK, f2), jnp.bfloat16)],
        compiler_params=params,
    )(adj, h1, b1, w2d, init, b2)
    return pooled[0]


def _solute_body(nmf_span, adj_ref, xa_ref, xn_ref, xw_ref, w1_ref, wf_ref,
                 bf_ref, b1_ref, b2_ref, w2d_ref, pool_ref):
    adj = adj_ref[...]
    w1 = w1_ref[...]
    wf = wf_ref[...]
    bf = bf_ref[...]
    h1_parts = []
    init_parts = []
    for x_ref in (xa_ref, xn_ref, xw_ref):
        for b in range(_B):
            xb = x_ref[b]
            h1_parts.append(
                jnp.dot(xb, w1, preferred_element_type=jnp.float32))
            init_parts.append(
                jnp.dot(xb, wf, preferred_element_type=jnp.float32) + bf)
    h1 = jnp.concatenate(h1_parts, axis=1)
    init = jnp.concatenate(init_parts, axis=1)
    s = _act(jnp.dot(adj, h1, preferred_element_type=jnp.float32)
             + b1_ref[...], nmf_span)
    h2 = jnp.dot(s, w2d_ref[...], preferred_element_type=jnp.float32)
    t = jnp.dot(adj, h2, preferred_element_type=jnp.float32) \
        + b2_ref[...] + init
    m = jnp.max(t, axis=0, keepdims=True)
    pool_ref[...] = jnp.broadcast_to(m, pool_ref.shape)


def _solute_pool(adj, xs, conv1_w, fc1_w, fc1_b, b1, b2, w2d, nmf_span):
    n = adj.shape[0]
    f1 = w2d.shape[0]
    f2 = w2d.shape[1]
    full = lambda shape: pl.BlockSpec(shape, lambda: (0,) * len(shape))
    pooled = pl.pallas_call(
        functools.partial(_solute_body, nmf_span),
        in_specs=[
            full((n, n)),
            full((_B, n, _NFEAT)), full((_B, n, _NFEAT)),
            full((_B, n, _NFEAT)),
            full((_NFEAT, _NHID)), full((_NFEAT, _NCLASS)),
            full((1, _NCLASS)), full((1, f1)), full((1, f2)),
            full((f1, f2)),
        ],
        out_specs=full((8, f2)),
        out_shape=jax.ShapeDtypeStruct((8, f2), jnp.float32),
        compiler_params=pltpu.CompilerParams(
            vmem_limit_bytes=64 * 1024 * 1024),
    )(adj, xs[0], xs[1], xs[2], conv1_w, fc1_w,
      fc1_b.reshape(1, _NCLASS), b1, b2, w2d)
    return pooled[0]


def _head_body(x_ref, w2_ref, b2_ref, w3_ref, b3_ref, w4_ref, b4_ref,
               w5_ref, b5_ref, out_ref):
    x = x_ref[...]
    x = jnp.maximum(jnp.dot(x, w2_ref[...], preferred_element_type=jnp.float32)
                    + b2_ref[...], 0.0)
    x = jnp.maximum(jnp.dot(x, w3_ref[...], preferred_element_type=jnp.float32)
                    + b3_ref[...], 0.0)
    x = jnp.maximum(jnp.dot(x, w4_ref[...], preferred_element_type=jnp.float32)
                    + b4_ref[...], 0.0)
    d = jnp.dot(x, w5_ref[...], preferred_element_type=jnp.float32) + b5_ref[...]
    out_ref[...] = d + jnp.zeros((8, 8), jnp.float32)


def _head(x8, fc2_w, fc2_b, fc3_w, fc3_b, fc4_w, fc4_b, fc5_w, fc5_b):
    full = lambda shape: pl.BlockSpec(shape, lambda: (0,) * len(shape))
    return pl.pallas_call(
        _head_body,
        in_specs=[
            full((8, 2 * _NCLASS)),
            full(fc2_w.shape), full((1, _NCLASS)),
            full(fc3_w.shape), full((1, 64)),
            full(fc4_w.shape), full((1, 32)),
            full(fc5_w.shape), full((1, 1)),
        ],
        out_specs=full((8, 8)),
        out_shape=jax.ShapeDtypeStruct((8, 8), jnp.float32),
    )(x8, fc2_w, fc2_b.reshape(1, -1), fc3_w, fc3_b.reshape(1, -1),
      fc4_w, fc4_b.reshape(1, -1), fc5_w, fc5_b.reshape(1, -1))


def kernel(solute_ACE, solvent_ACE, solute_adj, solvent_adj_ACE, solute_NMF,
           solvent_NMF, solvent_adj_NMF, solute_wat, solvent_wat,
           solvent_adj_wat, fc1_w, fc1_b, conv1_w, conv1_b, conv2_w, conv2_b,
           fc2_w, fc2_b, fc3_w, fc3_b, fc4_w, fc4_b, fc5_w, fc5_b):
    sv_h1, sv_init = {}, {}
    sv_h1['ACE'], sv_init['ACE'] = _feat([solvent_ACE], conv1_w, fc1_w, fc1_b)
    sv_h1['NMF'], sv_init['NMF'] = _feat([solvent_NMF], conv1_w, fc1_w, fc1_b)
    sv_h1['wat'], sv_init['wat'] = _feat([solvent_wat], conv1_w, fc1_w, fc1_b)

    b1_sv = jnp.tile(conv1_b, _B).reshape(1, -1)
    b2_sv = jnp.tile(conv2_b, _B).reshape(1, -1)
    b1_su = jnp.tile(conv1_b, 3 * _B).reshape(1, -1)
    b2_su = jnp.tile(conv2_b, 3 * _B).reshape(1, -1)
    w2_sv = jnp.kron(jnp.eye(_B, dtype=jnp.float32), conv2_w)
    w2_su = jnp.kron(jnp.eye(3 * _B, dtype=jnp.float32), conv2_w)

    p_su = _solute_pool(solute_adj, [solute_ACE, solute_NMF, solute_wat],
                        conv1_w, fc1_w, fc1_b, b1_su, b2_su, w2_su,
                        nmf_span=(_B * _NHID, 2 * _B * _NHID))
    p_ace = _gcn_pool(solvent_adj_ACE, sv_h1['ACE'], sv_init['ACE'],
                      b1_sv, b2_sv, w2_sv, nmf_span=None)
    p_nmf = _gcn_pool(solvent_adj_NMF, sv_h1['NMF'], sv_init['NMF'],
                      b1_sv, b2_sv, w2_sv, nmf_span=None)
    p_wat = _gcn_pool(solvent_adj_wat, sv_h1['wat'], sv_init['wat'],
                      b1_sv, b2_sv, w2_sv, nmf_span=None)

    c = _NCLASS
    rows = [
        jnp.concatenate([p_su[0 * c:1 * c], p_ace[0:c]]),
        jnp.concatenate([p_su[1 * c:2 * c], p_ace[c:2 * c]]),
        jnp.concatenate([p_su[2 * c:3 * c], p_nmf[0:c]]),
        jnp.concatenate([p_su[3 * c:4 * c], p_nmf[c:2 * c]]),
        jnp.concatenate([p_su[4 * c:5 * c], p_wat[0:c]]),
        jnp.concatenate([p_su[5 * c:6 * c], p_wat[c:2 * c]]),
    ]
    x8 = jnp.pad(jnp.stack(rows), ((0, 2), (0, 0)))
    out = _head(x8, fc2_w, fc2_b, fc3_w, fc3_b, fc4_w, fc4_b, fc5_w, fc5_b)
    return out[:6, :1]


# speedup vs baseline: 1.0835x; 1.0027x over previous
import functools

import jax
import jax.numpy as jnp
from jax.experimental import pallas as pl
from jax.experimental.pallas import tpu as pltpu

_NFEAT = 128
_NHID = 64
_NCLASS = 32
_B = 2

_ROW_BLK = 480


def _feat_body(n_sets, w1_ref, wf_ref, bf_ref, *refs):
    x_refs = refs[:n_sets]
    h1_ref, init_ref = refs[n_sets], refs[n_sets + 1]
    w1 = w1_ref[...]
    wf = wf_ref[...]
    bf = bf_ref[...]
    h1_parts = []
    init_parts = []
    for x_ref in x_refs:
        for b in range(_B):
            xb = x_ref[b]
            h1_parts.append(
                jnp.dot(xb, w1, preferred_element_type=jnp.float32))
            init_parts.append(
                jnp.dot(xb, wf, preferred_element_type=jnp.float32) + bf)
    h1_ref[...] = jnp.concatenate(h1_parts, axis=1).astype(jnp.bfloat16)
    init_ref[...] = jnp.concatenate(init_parts, axis=1)


def _feat(xs, conv1_w, fc1_w, fc1_b):
    n_sets = len(xs)
    n = xs[0].shape[1]
    grid = pl.cdiv(n, _ROW_BLK)
    return pl.pallas_call(
        functools.partial(_feat_body, n_sets),
        grid=(grid,),
        in_specs=[
            pl.BlockSpec((_NFEAT, _NHID), lambda i: (0, 0)),
            pl.BlockSpec((_NFEAT, _NCLASS), lambda i: (0, 0)),
            pl.BlockSpec((1, _NCLASS), lambda i: (0, 0)),
        ] + [
            pl.BlockSpec((_B, _ROW_BLK, _NFEAT), lambda i: (0, i, 0))
            for _ in range(n_sets)
        ],
        out_specs=[
            pl.BlockSpec((_ROW_BLK, n_sets * _B * _NHID), lambda i: (i, 0)),
            pl.BlockSpec((_ROW_BLK, n_sets * _B * _NCLASS), lambda i: (i, 0)),
        ],
        out_shape=[
            jax.ShapeDtypeStruct((n, n_sets * _B * _NHID), jnp.bfloat16),
            jax.ShapeDtypeStruct((n, n_sets * _B * _NCLASS), jnp.float32),
        ],
    )(conv1_w, fc1_w, fc1_b.reshape(1, _NCLASS), *xs)


def _act(t, nmf_span):
    if nmf_span is None:
        return jnp.maximum(t, 0.0)
    lo, hi = nmf_span
    col = jax.lax.broadcasted_iota(jnp.int32, t.shape, 1)
    keep_linear = (col >= lo) & (col < hi)
    return jnp.where(keep_linear, t, jnp.maximum(t, 0.0))


def _graph_body(n_rows, g1, adj_ref, h1_ref, b1_ref, w2_ref, init_ref,
                b2_ref, pool_ref, h2_scr):
    i = pl.program_id(0)

    @pl.when(i < g1)
    def _():
        s = _act(jnp.dot(adj_ref[...], h1_ref[...],
                         preferred_element_type=jnp.float32) + b1_ref[...],
                 None)
        h2_scr[pl.ds(i * _ROW_BLK, _ROW_BLK), :] = jnp.dot(
            s, w2_ref[...], preferred_element_type=jnp.float32
        ).astype(jnp.bfloat16)

    @pl.when(i >= g1)
    def _():
        k = i - g1
        h2 = h2_scr[...][0:n_rows, :]
        t = (
            jnp.dot(adj_ref[...], h2, preferred_element_type=jnp.float32)
            + b2_ref[...] + init_ref[...]
        )
        rows = jax.lax.broadcasted_iota(jnp.int32, t.shape, 0) + k * _ROW_BLK
        t = jnp.where(rows < n_rows, t, -jnp.inf)
        m8 = jnp.broadcast_to(jnp.max(t, axis=0, keepdims=True),
                              (8, t.shape[1]))

        @pl.when(k == 0)
        def _():
            pool_ref[...] = m8

        @pl.when(k > 0)
        def _():
            pool_ref[...] = jnp.maximum(pool_ref[...], m8)


def _gcn_pool(adj, h1, init, b1, b2, w2d, nmf_span):
    del nmf_span
    n = adj.shape[0]
    f1 = h1.shape[1]
    f2 = w2d.shape[1]
    g1 = pl.cdiv(n, _ROW_BLK)
    params = pltpu.CompilerParams(vmem_limit_bytes=64 * 1024 * 1024)
    phase_blk = lambda i: (jnp.where(i < g1, i, i - g1), 0)
    pooled = pl.pallas_call(
        functools.partial(_graph_body, n, g1),
        grid=(2 * g1,),
        in_specs=[
            pl.BlockSpec((_ROW_BLK, n), phase_blk),
            pl.BlockSpec((n, f1), lambda i: (0, 0)),
            pl.BlockSpec((1, f1), lambda i: (0, 0)),
            pl.BlockSpec((f1, f2), lambda i: (0, 0)),
            pl.BlockSpec((_ROW_BLK, f2), phase_blk),
            pl.BlockSpec((1, f2), lambda i: (0, 0)),
        ],
        out_specs=pl.BlockSpec((8, f2), lambda i: (0, 0)),
        out_shape=jax.ShapeDtypeStruct((8, f2), jnp.float32),
        scratch_shapes=[pltpu.VMEM((g1 * _ROW_BLK, f2), jnp.bfloat16)],
        compiler_params=params,
    )(adj, h1, b1, w2d, init, b2)
    return pooled[0]


def _solute_body(nmf_span, adj_ref, xa_ref, xn_ref, xw_ref, w1_ref, wf_ref,
                 bf_ref, b1_ref, b2_ref, w2d_ref, pool_ref):
    adj = adj_ref[...]
    w1 = w1_ref[...]
    wf = wf_ref[...]
    bf = bf_ref[...]
    h1_parts = []
    init_parts = []
    for x_ref in (xa_ref, xn_ref, xw_ref):
        for b in range(_B):
            xb = x_ref[b]
            h1_parts.append(
                jnp.dot(xb, w1, preferred_element_type=jnp.float32))
            init_parts.append(
                jnp.dot(xb, wf, preferred_element_type=jnp.float32) + bf)
    h1 = jnp.concatenate(h1_parts, axis=1)
    init = jnp.concatenate(init_parts, axis=1)
    s = _act(jnp.dot(adj, h1, preferred_element_type=jnp.float32)
             + b1_ref[...], nmf_span)
    h2 = jnp.dot(s, w2d_ref[...], preferred_element_type=jnp.float32)
    t = jnp.dot(adj, h2, preferred_element_type=jnp.float32) \
        + b2_ref[...] + init
    m = jnp.max(t, axis=0, keepdims=True)
    pool_ref[...] = jnp.broadcast_to(m, pool_ref.shape)


def _solute_pool(adj, xs, conv1_w, fc1_w, fc1_b, b1, b2, w2d, nmf_span):
    n = adj.shape[0]
    f1 = w2d.shape[0]
    f2 = w2d.shape[1]
    full = lambda shape: pl.BlockSpec(shape, lambda: (0,) * len(shape))
    pooled = pl.pallas_call(
        functools.partial(_solute_body, nmf_span),
        in_specs=[
            full((n, n)),
            full((_B, n, _NFEAT)), full((_B, n, _NFEAT)),
            full((_B, n, _NFEAT)),
            full((_NFEAT, _NHID)), full((_NFEAT, _NCLASS)),
            full((1, _NCLASS)), full((1, f1)), full((1, f2)),
            full((f1, f2)),
        ],
        out_specs=full((8, f2)),
        out_shape=jax.ShapeDtypeStruct((8, f2), jnp.float32),
        compiler_params=pltpu.CompilerParams(
            vmem_limit_bytes=64 * 1024 * 1024),
    )(adj, xs[0], xs[1], xs[2], conv1_w, fc1_w,
      fc1_b.reshape(1, _NCLASS), b1, b2, w2d)
    return pooled[0]


def _head_body(x_ref, w2_ref, b2_ref, w3_ref, b3_ref, w4_ref, b4_ref,
               w5_ref, b5_ref, out_ref):
    x = x_ref[...]
    x = jnp.maximum(jnp.dot(x, w2_ref[...], preferred_element_type=jnp.float32)
                    + b2_ref[...], 0.0)
    x = jnp.maximum(jnp.dot(x, w3_ref[...], preferred_element_type=jnp.float32)
                    + b3_ref[...], 0.0)
    x = jnp.maximum(jnp.dot(x, w4_ref[...], preferred_element_type=jnp.float32)
                    + b4_ref[...], 0.0)
    d = jnp.dot(x, w5_ref[...], preferred_element_type=jnp.float32) + b5_ref[...]
    out_ref[...] = d + jnp.zeros((8, 8), jnp.float32)


def _head(x8, fc2_w, fc2_b, fc3_w, fc3_b, fc4_w, fc4_b, fc5_w, fc5_b):
    full = lambda shape: pl.BlockSpec(shape, lambda: (0,) * len(shape))
    return pl.pallas_call(
        _head_body,
        in_specs=[
            full((8, 2 * _NCLASS)),
            full(fc2_w.shape), full((1, _NCLASS)),
            full(fc3_w.shape), full((1, 64)),
            full(fc4_w.shape), full((1, 32)),
            full(fc5_w.shape), full((1, 1)),
        ],
        out_specs=full((8, 8)),
        out_shape=jax.ShapeDtypeStruct((8, 8), jnp.float32),
    )(x8, fc2_w, fc2_b.reshape(1, -1), fc3_w, fc3_b.reshape(1, -1),
      fc4_w, fc4_b.reshape(1, -1), fc5_w, fc5_b.reshape(1, -1))


def kernel(solute_ACE, solvent_ACE, solute_adj, solvent_adj_ACE, solute_NMF,
           solvent_NMF, solvent_adj_NMF, solute_wat, solvent_wat,
           solvent_adj_wat, fc1_w, fc1_b, conv1_w, conv1_b, conv2_w, conv2_b,
           fc2_w, fc2_b, fc3_w, fc3_b, fc4_w, fc4_b, fc5_w, fc5_b):
    sv_h1, sv_init = {}, {}
    sv_h1['ACE'], sv_init['ACE'] = _feat([solvent_ACE], conv1_w, fc1_w, fc1_b)
    sv_h1['NMF'], sv_init['NMF'] = _feat([solvent_NMF], conv1_w, fc1_w, fc1_b)
    sv_h1['wat'], sv_init['wat'] = _feat([solvent_wat], conv1_w, fc1_w, fc1_b)

    b1_sv = jnp.tile(conv1_b, _B).reshape(1, -1)
    b2_sv = jnp.tile(conv2_b, _B).reshape(1, -1)
    b1_su = jnp.tile(conv1_b, 3 * _B).reshape(1, -1)
    b2_su = jnp.tile(conv2_b, 3 * _B).reshape(1, -1)
    w2_sv = jnp.kron(jnp.eye(_B, dtype=jnp.float32), conv2_w)
    w2_su = jnp.kron(jnp.eye(3 * _B, dtype=jnp.float32), conv2_w)

    p_su = _solute_pool(solute_adj, [solute_ACE, solute_NMF, solute_wat],
                        conv1_w, fc1_w, fc1_b, b1_su, b2_su, w2_su,
                        nmf_span=(_B * _NHID, 2 * _B * _NHID))
    p_ace = _gcn_pool(solvent_adj_ACE, sv_h1['ACE'], sv_init['ACE'],
                      b1_sv, b2_sv, w2_sv, nmf_span=None)
    p_nmf = _gcn_pool(solvent_adj_NMF, sv_h1['NMF'], sv_init['NMF'],
                      b1_sv, b2_sv, w2_sv, nmf_span=None)
    p_wat = _gcn_pool(solvent_adj_wat, sv_h1['wat'], sv_init['wat'],
                      b1_sv, b2_sv, w2_sv, nmf_span=None)

    c = _NCLASS
    rows = [
        jnp.concatenate([p_su[0 * c:1 * c], p_ace[0:c]]),
        jnp.concatenate([p_su[1 * c:2 * c], p_ace[c:2 * c]]),
        jnp.concatenate([p_su[2 * c:3 * c], p_nmf[0:c]]),
        jnp.concatenate([p_su[3 * c:4 * c], p_nmf[c:2 * c]]),
        jnp.concatenate([p_su[4 * c:5 * c], p_wat[0:c]]),
        jnp.concatenate([p_su[5 * c:6 * c], p_wat[c:2 * c]]),
    ]
    x8 = jnp.pad(jnp.stack(rows), ((0, 2), (0, 0)))
    out = _head(x8, fc2_w, fc2_b, fc3_w, fc3_b, fc4_w, fc4_b, fc5_w, fc5_b)
    return out[:6, :1]
